# baseline (speedup 1.0000x reference)
import functools

import jax
import jax.numpy as jnp
from jax import lax
from jax.experimental import pallas as pl
from jax.experimental.pallas import tpu as pltpu
from jax.experimental.pallas import tpu_sc as plsc

_NC = 2
_NS = 16
_L = 16
_NW = _NC * _NS
_NBUF = 4


def _make_fq(N, C, S, KB, NCH):
    SP = (S + 15) // 8 * 8
    NB = N // 128
    HB = NB // 2
    CHI = C // 8

    mesh = plsc.VectorSubcoreMesh(core_axis_name="c", subcore_axis_name="s")

    @functools.partial(
        pl.kernel,
        mesh=mesh,
        out_type=jax.ShapeDtypeStruct((CHI, NB, 8, 128), jnp.float32),
        compiler_params=pltpu.CompilerParams(needs_layout_passes=False),
        scratch_types=(
            [
                pltpu.VMEM((SP,), jnp.float32),
                pltpu.VMEM((SP,), jnp.float32),
                pltpu.VMEM((_L,), jnp.float32),
                pltpu.VMEM((_L,), jnp.float32),
            ]
            + [pltpu.VMEM((KB, 128), jnp.float32) for _ in range(2 * _NBUF)]
            + [pltpu.SemaphoreType.DMA for _ in range(2 * _NBUF)]
        ),
    )
    def fq(x_hbm, tab_hbm, tab1_hbm, xmn_hbm, xmx_hbm, o_hbm,
           tab_v, tab1_v, xmn_v, xmx_v, *bufs_and_sems):
        ins = bufs_and_sems[0:_NBUF]
        outs = bufs_and_sems[_NBUF:2 * _NBUF]
        sin = bufs_and_sems[2 * _NBUF:3 * _NBUF]
        sout = bufs_and_sems[3 * _NBUF:4 * _NBUF]

        wid = lax.axis_index("s") * _NC + lax.axis_index("c")
        col = wid // 2
        half = wid % 2
        chi = col // 8
        clo = col % 8
        blk0 = half * HB

        pltpu.sync_copy(tab_hbm.at[pl.ds(col * S, SP)], tab_v)
        pltpu.sync_copy(tab1_hbm.at[pl.ds(col * S, SP)], tab1_v)
        pltpu.sync_copy(xmn_hbm, xmn_v)
        pltpu.sync_copy(xmx_hbm, xmx_v)

        def in_copy(k, p):
            return pltpu.make_async_copy(
                x_hbm.at[chi, pl.ds(blk0 + k * KB, KB), clo, :],
                ins[p], sin[p])

        def out_copy(k, p):
            return pltpu.make_async_copy(
                outs[p], o_hbm.at[chi, pl.ds(blk0 + k * KB, KB), clo, :],
                sout[p])

        cvec = jnp.broadcast_to(col, (_L,)).astype(jnp.int32)
        xmn = plsc.load_gather(xmn_v, [cvec])
        xmx = plsc.load_gather(xmx_v, [cvec])
        scl = float(S - 1) / (xmx - xmn)
        tmax = float(S - 1)
        bmax = float(S - 2)

        def chunk(src, dst):
            @plsc.parallel_loop(0, KB, unroll=4)
            def body(blk):
                for j in range(8):
                    x = src[blk, pl.ds(j * _L, _L)]
                    t = (x - xmn) * scl
                    t = jnp.minimum(jnp.maximum(t, 0.0), tmax)
                    i0 = jnp.minimum(t, bmax).astype(jnp.int32)
                    fr = t - i0.astype(jnp.float32)
                    y0 = plsc.load_gather(tab_v, [i0])
                    y1 = plsc.load_gather(tab1_v, [i0])
                    dst[blk, pl.ds(j * _L, _L)] = y0 + fr * (y1 - y0)

        for p in range(_NBUF):
            in_copy(p, p).start()

        def outer(kk, carry):
            for p in range(_NBUF):
                k = kk * _NBUF + p
                in_copy(k, p).wait()

                @pl.when(kk >= 1)
                def _():
                    out_copy(k - _NBUF, p).wait()

                chunk(ins[p], outs[p])
                out_copy(k, p).start()

                @pl.when(kk < NCH // _NBUF - 1)
                def _():
                    in_copy(k + _NBUF, p).start()
            return carry
        lax.fori_loop(0, NCH // _NBUF, outer, 0)
        for p in range(_NBUF):
            out_copy(NCH - _NBUF + p, p).wait()

    return fq


def kernel(X, y_values, x_min, x_max):
    N, C = X.shape
    S = y_values.shape[1]
    KB = 64
    HB = N // 128 // 2
    while HB % (KB * _NBUF):
        KB //= 2
    NCH = HB // KB
    fq = _make_fq(N, C, S, KB, NCH)
    x4 = X.reshape(N // 128, 128, C // 8, 8).transpose(2, 0, 3, 1)
    SP = (S + 15) // 8 * 8
    pad = C * SP - C * S
    yv0 = jnp.concatenate([y_values.reshape(C * S), jnp.zeros((pad,), jnp.float32)])
    yv1 = jnp.concatenate(
        [y_values[:, 1:], y_values[:, -1:]], axis=1).reshape(C * S)
    yv1 = jnp.concatenate([yv1, jnp.zeros((pad,), jnp.float32)])
    out4 = fq(x4, yv0, yv1, x_min, x_max)
    return out4.transpose(1, 3, 0, 2).reshape(N, C)
